# Initial kernel scaffold; baseline (speedup 1.0000x reference)
import jax
import jax.numpy as jnp
from jax.experimental import pallas as pl


def kernel(values, prefix_sum):
    raise NotImplementedError("write your pallas kernel here")


# speedup vs baseline: 5.6348x; 5.6348x over previous
import functools

import jax
import jax.numpy as jnp
from jax import lax
from jax.experimental import pallas as pl
from jax.experimental.pallas import tpu as pltpu
from jax.experimental.pallas import tpu_sc as plsc

N = 32768
D = 512
B = 16
NC = 2
NS = 16
L = 16
CPC = D // NC
KV = CPC // L
RPT = N // NS
CH = 128
NCH = RPT // CH

_mesh = plsc.VectorSubcoreMesh(core_axis_name="c", subcore_axis_name="s")


@functools.partial(
    pl.kernel,
    mesh=_mesh,
    out_type=jax.ShapeDtypeStruct((B, D), jnp.float32),
    scratch_types=[
        pltpu.VMEM((CH, CPC), jnp.float32),
        pltpu.VMEM((CH, CPC), jnp.float32),
        pltpu.VMEM((B, CPC), jnp.float32),
        pltpu.VMEM((NS, CPC), jnp.float32),
        pltpu.VMEM((32,), jnp.int32),
        pltpu.VMEM_SHARED((B, NS, CPC), jnp.float32),
        pltpu.SemaphoreType.DMA,
        pltpu.SemaphoreType.DMA,
    ],
)
def _jagged_max(values_hbm, ps_hbm, out_hbm,
                buf0, buf1, partial, mbuf, ps_v, shared, sem0, sem1):
    cid = lax.axis_index("c")
    sid = lax.axis_index("s")
    c0 = cid * CPC
    row0 = sid * RPT

    pltpu.sync_copy(ps_hbm, ps_v)
    pvec0 = ps_v[pl.ds(0, L)]
    pvec1 = ps_v[pl.ds(L, L)]
    ps_s = [pvec0[i] for i in range(L)] + [pvec1[0]]

    neg = jnp.full((L,), -jnp.inf, jnp.float32)
    for s in range(B):
        for k in range(KV):
            partial[s, pl.ds(k * L, L)] = neg

    def start(j, buf, sem):
        pltpu.async_copy(
            values_hbm.at[pl.ds(row0 + j * CH, CH), pl.ds(c0, CPC)], buf, sem)

    def wait(buf, sem):
        pltpu.make_async_copy(
            values_hbm.at[pl.ds(row0, CH), pl.ds(c0, CPC)], buf, sem).wait()

    start(0, buf0, sem0)
    start(1, buf1, sem1)

    def process(j, buf):
        chunk_lo = row0 + j * CH
        for s in range(B):
            a = jnp.maximum(ps_s[s], chunk_lo) - chunk_lo
            b = jnp.minimum(ps_s[s + 1], chunk_lo + CH) - chunk_lo

            @pl.when(b > a)
            def _():
                acc0 = tuple(partial[s, pl.ds(k * L, L)] for k in range(KV))

                def rbody(r, acc):
                    return tuple(
                        jnp.maximum(acc[k], buf[r, pl.ds(k * L, L)])
                        for k in range(KV))

                acc = lax.fori_loop(a, b, rbody, acc0)
                for k in range(KV):
                    partial[s, pl.ds(k * L, L)] = acc[k]

    def loop_body(jj, carry):
        j = 2 * jj
        wait(buf0, sem0)
        process(j, buf0)

        @pl.when(j + 2 < NCH)
        def _():
            start(j + 2, buf0, sem0)

        wait(buf1, sem1)
        process(j + 1, buf1)

        @pl.when(j + 3 < NCH)
        def _():
            start(j + 3, buf1, sem1)

        return carry

    lax.fori_loop(0, NCH // 2, loop_body, 0)

    for s in range(B):
        pltpu.sync_copy(partial.at[s], shared.at[s, sid])
    plsc.subcore_barrier()
    pltpu.sync_copy(shared.at[sid], mbuf)
    for k in range(KV):
        acc = mbuf[0, pl.ds(k * L, L)]
        for t in range(1, NS):
            acc = jnp.maximum(acc, mbuf[t, pl.ds(k * L, L)])
        partial[0, pl.ds(k * L, L)] = acc
    pltpu.sync_copy(partial.at[0], out_hbm.at[sid, pl.ds(c0, CPC)])


@jax.jit
def kernel(values, prefix_sum):
    ps = jnp.pad(prefix_sum, (0, 32 - (B + 1)), mode="edge")
    return _jagged_max(values, ps)
